# baseline (speedup 1.0000x reference)
import functools
import math

import jax
import jax.numpy as jnp
from jax import lax
from jax.experimental import pallas as pl
from jax.experimental.pallas import tpu as pltpu
from jax.experimental.pallas import tpu_sc as plsc

N = 50000
E = 800000
NSEG = 14
NC = 2
NS = 16
LANES = 16

NPAD = 50176
ROW_W = 128
EROWS = E // ROW_W
GROUP = 16
FULLG = (EROWS // GROUP)
TAILB = FULLG * GROUP
ROUNDS = 7
SLICE = NPAD // NS

_mesh = plsc.VectorSubcoreMesh(core_axis_name="c", subcore_axis_name="s")
_sc_params = pltpu.CompilerParams(needs_layout_passes=False)


def _zero_fill(vbuf):
    zv = jnp.zeros((LANES,), jnp.float32)

    def zbody(i, carry):
        vbuf[pl.ds(i * LANES, LANES)] = zv
        return carry

    lax.fori_loop(0, SLICE // LANES, zbody, 0)


def _group_base(wid, r):
    gid = wid + (NC * NS) * r
    valid = gid * GROUP < TAILB
    base = pl.multiple_of(jnp.where(valid, gid, 0) * GROUP, GROUP)
    vmask = valid.astype(jnp.float32)
    return base, vmask


def _edge_pass1(x_hbm, ei_hbm, w_hbm, outa_hbm, outb_hbm,
                xv, src_st, dst_st, w_st, msg_st, vbuf, acc, sems):
    c = lax.axis_index("c")
    s = lax.axis_index("s")
    wid = c * NS + s
    pltpu.sync_copy(x_hbm, xv)
    _zero_fill(vbuf)
    pltpu.sync_copy(vbuf, acc.at[pl.ds(s * SLICE, SLICE)])
    plsc.subcore_barrier()

    def stage(r, b):
        base, vmask = _group_base(wid, r)
        rows = pl.ds(base, GROUP)
        cps = (pltpu.async_copy(ei_hbm.at[0, rows], src_st.at[b], sems.at[2 * b]),
               pltpu.async_copy(ei_hbm.at[1, rows], dst_st.at[b], sems.at[2 * b]),
               pltpu.async_copy(w_hbm.at[rows], w_st.at[b], sems.at[2 * b]))
        return cps, vmask

    def fire(b, vmask):
        cps = []
        for j in range(GROUP):
            for k in range(ROW_W // LANES):
                sl = pl.ds(k * LANES, LANES)
                idx = src_st[b, j, sl]
                xg = plsc.load_gather(xv, [idx])
                msg_st[b, j, sl] = xg * w_st[b, j, sl] * vmask
            cps.append(pltpu.async_copy(
                msg_st.at[b, j], acc.at[dst_st.at[b, j]],
                sems.at[2 * b + 1], add=True))
        return cps

    st0, m0 = stage(0, 0)
    st1, m1 = stage(1, 1)

    def body(i, carry):
        m0, m1 = carry
        r = 2 * i
        for cp in st0:
            cp.wait()
        sc0 = fire(0, m0)
        for cp in st1:
            cp.wait()
        sc1 = fire(1, m1)
        for cp in sc0:
            cp.wait()
        _, nm0 = stage(r + 2, 0)
        for cp in sc1:
            cp.wait()
        _, nm1 = stage(r + 3, 1)
        return (nm0, nm1)

    lax.fori_loop(0, ROUNDS, body, (m0, m1))
    for b in range(2):
        pltpu.make_async_copy(ei_hbm.at[0, pl.ds(0, GROUP)],
                              src_st.at[b], sems.at[2 * b]).wait()
        pltpu.make_async_copy(ei_hbm.at[1, pl.ds(0, GROUP)],
                              dst_st.at[b], sems.at[2 * b]).wait()
        pltpu.make_async_copy(w_hbm.at[pl.ds(0, GROUP)],
                              w_st.at[b], sems.at[2 * b]).wait()

    for tw, tbase, trows in ((NC * NS - 1, TAILB, 8), (NC * NS - 2, TAILB + 8, 2)):
        @pl.when(wid == tw)
        def _(tbase=tbase, trows=trows):
            rows = pl.ds(tbase, trows)
            pltpu.sync_copy(ei_hbm.at[0, rows], src_st.at[0, pl.ds(0, trows)])
            pltpu.sync_copy(ei_hbm.at[1, rows], dst_st.at[0, pl.ds(0, trows)])
            pltpu.sync_copy(w_hbm.at[rows], w_st.at[0, pl.ds(0, trows)])
            for j in range(trows):
                for k in range(ROW_W // LANES):
                    sl = pl.ds(k * LANES, LANES)
                    idx = src_st[0, j, sl]
                    xg = plsc.load_gather(xv, [idx])
                    msg_st[0, j, sl] = xg * w_st[0, j, sl]
                pltpu.sync_copy(msg_st.at[0, j], acc.at[dst_st.at[0, j]],
                                add=True)

    plsc.subcore_barrier()
    osl = pl.ds(s * SLICE, SLICE)
    pltpu.sync_copy(acc.at[osl], vbuf)

    @pl.when(c == 0)
    def _():
        pltpu.sync_copy(vbuf, outa_hbm.at[osl])

    @pl.when(c == 1)
    def _():
        pltpu.sync_copy(vbuf, outb_hbm.at[osl])


def _edge_pass2(s1a_hbm, s1b_hbm, ei_hbm, w_hbm,
                sua_hbm, sub_hbm, sva_hbm, svb_hbm,
                t0, t1, src_st, dst_st, w_st, mu_st, mv_st, vbuf,
                accu, accv, sems):
    c = lax.axis_index("c")
    s = lax.axis_index("s")
    wid = c * NS + s
    pltpu.sync_copy(s1a_hbm, t0)
    pltpu.sync_copy(s1b_hbm, t1)
    _zero_fill(vbuf)
    pltpu.sync_copy(vbuf, accu.at[pl.ds(s * SLICE, SLICE)])
    pltpu.sync_copy(vbuf, accv.at[pl.ds(s * SLICE, SLICE)])
    plsc.subcore_barrier()

    def stage(r, b):
        base, vmask = _group_base(wid, r)
        rows = pl.ds(base, GROUP)
        cps = (pltpu.async_copy(ei_hbm.at[0, rows], src_st.at[b], sems.at[2 * b]),
               pltpu.async_copy(ei_hbm.at[1, rows], dst_st.at[b], sems.at[2 * b]),
               pltpu.async_copy(w_hbm.at[rows], w_st.at[b], sems.at[2 * b]))
        return cps, vmask

    def fire(b, vmask):
        cps = []
        for j in range(GROUP):
            for k in range(ROW_W // LANES):
                sl = pl.ds(k * LANES, LANES)
                idx = src_st[b, j, sl]
                sval = plsc.load_gather(t0, [idx]) + plsc.load_gather(t1, [idx])
                wv = w_st[b, j, sl] * vmask
                mu_st[b, j, sl] = jnp.maximum(sval, 0.0) * wv
                mv_st[b, j, sl] = jnp.maximum(-sval, 0.0) * wv
            cps.append(pltpu.async_copy(
                mu_st.at[b, j], accu.at[dst_st.at[b, j]],
                sems.at[2 * b + 1], add=True))
            cps.append(pltpu.async_copy(
                mv_st.at[b, j], accv.at[dst_st.at[b, j]],
                sems.at[2 * b + 1], add=True))
        return cps

    st0, m0 = stage(0, 0)
    st1, m1 = stage(1, 1)

    def body(i, carry):
        m0, m1 = carry
        r = 2 * i
        for cp in st0:
            cp.wait()
        sc0 = fire(0, m0)
        for cp in st1:
            cp.wait()
        sc1 = fire(1, m1)
        for cp in sc0:
            cp.wait()
        _, nm0 = stage(r + 2, 0)
        for cp in sc1:
            cp.wait()
        _, nm1 = stage(r + 3, 1)
        return (nm0, nm1)

    lax.fori_loop(0, ROUNDS, body, (m0, m1))
    for b in range(2):
        pltpu.make_async_copy(ei_hbm.at[0, pl.ds(0, GROUP)],
                              src_st.at[b], sems.at[2 * b]).wait()
        pltpu.make_async_copy(ei_hbm.at[1, pl.ds(0, GROUP)],
                              dst_st.at[b], sems.at[2 * b]).wait()
        pltpu.make_async_copy(w_hbm.at[pl.ds(0, GROUP)],
                              w_st.at[b], sems.at[2 * b]).wait()

    for tw, tbase, trows in ((NC * NS - 1, TAILB, 8), (NC * NS - 2, TAILB + 8, 2)):
        @pl.when(wid == tw)
        def _(tbase=tbase, trows=trows):
            rows = pl.ds(tbase, trows)
            pltpu.sync_copy(ei_hbm.at[0, rows], src_st.at[0, pl.ds(0, trows)])
            pltpu.sync_copy(ei_hbm.at[1, rows], dst_st.at[0, pl.ds(0, trows)])
            pltpu.sync_copy(w_hbm.at[rows], w_st.at[0, pl.ds(0, trows)])
            for j in range(trows):
                for k in range(ROW_W // LANES):
                    sl = pl.ds(k * LANES, LANES)
                    idx = src_st[0, j, sl]
                    sval = (plsc.load_gather(t0, [idx])
                            + plsc.load_gather(t1, [idx]))
                    wv = w_st[0, j, sl]
                    mu_st[0, j, sl] = jnp.maximum(sval, 0.0) * wv
                    mv_st[0, j, sl] = jnp.maximum(-sval, 0.0) * wv
                pltpu.sync_copy(mu_st.at[0, j], accu.at[dst_st.at[0, j]],
                                add=True)
                pltpu.sync_copy(mv_st.at[0, j], accv.at[dst_st.at[0, j]],
                                add=True)

    plsc.subcore_barrier()
    osl = pl.ds(s * SLICE, SLICE)
    pltpu.sync_copy(accu.at[osl], vbuf)

    @pl.when(c == 0)
    def _():
        pltpu.sync_copy(vbuf, sua_hbm.at[osl])

    @pl.when(c == 1)
    def _():
        pltpu.sync_copy(vbuf, sub_hbm.at[osl])

    pltpu.sync_copy(accv.at[osl], vbuf)

    @pl.when(c == 0)
    def _():
        pltpu.sync_copy(vbuf, sva_hbm.at[osl])

    @pl.when(c == 1)
    def _():
        pltpu.sync_copy(vbuf, svb_hbm.at[osl])


_pass1 = pl.kernel(
    _edge_pass1,
    out_type=(jax.ShapeDtypeStruct((NPAD,), jnp.float32),
              jax.ShapeDtypeStruct((NPAD,), jnp.float32)),
    mesh=_mesh,
    compiler_params=_sc_params,
    scratch_types=[
        pltpu.VMEM((N,), jnp.float32),
        pltpu.VMEM((2, GROUP, ROW_W), jnp.int32),
        pltpu.VMEM((2, GROUP, ROW_W), jnp.int32),
        pltpu.VMEM((2, GROUP, ROW_W), jnp.float32),
        pltpu.VMEM((2, GROUP, ROW_W), jnp.float32),
        pltpu.VMEM((SLICE,), jnp.float32),
        pltpu.VMEM_SHARED((NPAD,), jnp.float32),
        pltpu.SemaphoreType.DMA((4,)),
    ],
)

_pass2 = pl.kernel(
    _edge_pass2,
    out_type=(jax.ShapeDtypeStruct((NPAD,), jnp.float32),
              jax.ShapeDtypeStruct((NPAD,), jnp.float32),
              jax.ShapeDtypeStruct((NPAD,), jnp.float32),
              jax.ShapeDtypeStruct((NPAD,), jnp.float32)),
    mesh=_mesh,
    compiler_params=_sc_params,
    scratch_types=[
        pltpu.VMEM((NPAD,), jnp.float32),
        pltpu.VMEM((NPAD,), jnp.float32),
        pltpu.VMEM((2, GROUP, ROW_W), jnp.int32),
        pltpu.VMEM((2, GROUP, ROW_W), jnp.int32),
        pltpu.VMEM((2, GROUP, ROW_W), jnp.float32),
        pltpu.VMEM((2, GROUP, ROW_W), jnp.float32),
        pltpu.VMEM((2, GROUP, ROW_W), jnp.float32),
        pltpu.VMEM((SLICE,), jnp.float32),
        pltpu.VMEM_SHARED((NPAD,), jnp.float32),
        pltpu.VMEM_SHARED((NPAD,), jnp.float32),
        pltpu.SemaphoreType.DMA((4,)),
    ],
)


def _epilogue(sua_ref, sub_ref, sva_ref, svb_ref,
              w1_ref, w2_ref, b2_ref, wl_ref, bl_ref, out_ref):
    su = sua_ref[...] + sub_ref[...]
    sv = sva_ref[...] + svb_ref[...]
    w1 = w1_ref[...]
    w2 = w2_ref[...]
    alpha = jnp.dot(jnp.maximum(w1, 0.0), w2,
                    preferred_element_type=jnp.float32)
    beta = jnp.dot(jnp.maximum(-w1, 0.0), w2,
                   preferred_element_type=jnp.float32)
    rows = NPAD // 128
    i = (lax.broadcasted_iota(jnp.int32, (rows, 128), 0) * 128
         + lax.broadcasted_iota(jnp.int32, (rows, 128), 1))
    seg = (i * NSEG) // N
    gain = jnp.zeros((rows, 128), jnp.float32)
    for sgi in range(NSEG):
        cnt = -((-(sgi + 1) * N) // NSEG) - -((-sgi * N) // NSEG)
        gain = jnp.where(seg == sgi, wl_ref[sgi, 0] * (1.0 / cnt), gain)
    gain = jnp.where(i < N, gain, 0.0)
    lane = lax.broadcasted_iota(jnp.int32, (1, 128), 1)
    vec = jnp.zeros((1, 128), jnp.float32)
    for ci in range(32):
        h2c = jnp.maximum(su * alpha[0, ci] + sv * beta[0, ci] + b2_ref[0, ci],
                          0.0)
        vec = jnp.where(lane == ci, jnp.sum(h2c * gain), vec)
    out_ref[...] = 1.0 / (1.0 + jnp.exp(-(vec + bl_ref[0, 0])))


_epilogue_call = pl.pallas_call(
    _epilogue,
    out_shape=jax.ShapeDtypeStruct((1, 128), jnp.float32),
)


@jax.jit
def kernel(x, edge_index, edge_weight, W1, b1, W2, b2, Wl, bl):
    ei3 = edge_index.reshape(2, EROWS, ROW_W)
    w2e = edge_weight.reshape(EROWS, ROW_W)

    s1a, s1b = _pass1(x.reshape(N), ei3, w2e)
    sua, sub, sva, svb = _pass2(s1a, s1b, ei3, w2e)

    r = NPAD // 128
    vec = _epilogue_call(sua.reshape(r, 128), sub.reshape(r, 128),
                         sva.reshape(r, 128), svb.reshape(r, 128),
                         W1, W2, b2.reshape(1, 32), Wl, bl.reshape(1, 1))
    return vec[0, :32].reshape(32, 1)


# speedup vs baseline: 1.0647x; 1.0647x over previous
import functools
import math

import jax
import jax.numpy as jnp
from jax import lax
from jax.experimental import pallas as pl
from jax.experimental.pallas import tpu as pltpu
from jax.experimental.pallas import tpu_sc as plsc

N = 50000
E = 800000
NSEG = 14
NC = 2
NS = 16
LANES = 16

NPAD = 50176
ROW_W = 128
EROWS = E // ROW_W
GROUP = 8
FULLG = (EROWS // GROUP)
TAILB = FULLG * GROUP
ROUNDS = 13
SLICE = NPAD // NS

_mesh = plsc.VectorSubcoreMesh(core_axis_name="c", subcore_axis_name="s")
_sc_params = pltpu.CompilerParams(needs_layout_passes=False)


def _zero_fill(vbuf):
    zv = jnp.zeros((LANES,), jnp.float32)

    def zbody(i, carry):
        vbuf[pl.ds(i * LANES, LANES)] = zv
        return carry

    lax.fori_loop(0, SLICE // LANES, zbody, 0)


def _group_base(wid, r):
    gid = wid + (NC * NS) * r
    valid = gid * GROUP < TAILB
    base = pl.multiple_of(jnp.where(valid, gid, 0) * GROUP, GROUP)
    vmask = valid.astype(jnp.float32)
    return base, vmask


def _edge_pass1(x_hbm, ei_hbm, w_hbm, outa_hbm, outb_hbm,
                xv, src_st, dst_st, w_st, msg_st, vbuf, acc, sems):
    c = lax.axis_index("c")
    s = lax.axis_index("s")
    wid = c * NS + s
    pltpu.sync_copy(x_hbm, xv)
    _zero_fill(vbuf)
    pltpu.sync_copy(vbuf, acc.at[pl.ds(s * SLICE, SLICE)])
    plsc.subcore_barrier()

    def stage(r, b):
        base, vmask = _group_base(wid, r)
        rows = pl.ds(base, GROUP)
        cps = (pltpu.async_copy(ei_hbm.at[0, rows], src_st.at[b], sems.at[2 * b]),
               pltpu.async_copy(ei_hbm.at[1, rows], dst_st.at[b], sems.at[2 * b]),
               pltpu.async_copy(w_hbm.at[rows], w_st.at[b], sems.at[2 * b]))
        return cps, vmask

    def fire(b, vmask):
        cps = []
        for j in range(GROUP):
            for k in range(ROW_W // LANES):
                sl = pl.ds(k * LANES, LANES)
                idx = src_st[b, j, sl]
                xg = plsc.load_gather(xv, [idx])
                msg_st[b, j, sl] = xg * w_st[b, j, sl] * vmask
            cps.append(pltpu.async_copy(
                msg_st.at[b, j], acc.at[dst_st.at[b, j]],
                sems.at[2 * b + 1], add=True))
        return cps

    st0, m0 = stage(0, 0)
    st1, m1 = stage(1, 1)

    def body(i, carry):
        m0, m1 = carry
        r = 2 * i
        for cp in st0:
            cp.wait()
        sc0 = fire(0, m0)
        for cp in st1:
            cp.wait()
        sc1 = fire(1, m1)
        for cp in sc0:
            cp.wait()
        _, nm0 = stage(r + 2, 0)
        for cp in sc1:
            cp.wait()
        _, nm1 = stage(r + 3, 1)
        return (nm0, nm1)

    lax.fori_loop(0, ROUNDS, body, (m0, m1))
    for b in range(2):
        pltpu.make_async_copy(ei_hbm.at[0, pl.ds(0, GROUP)],
                              src_st.at[b], sems.at[2 * b]).wait()
        pltpu.make_async_copy(ei_hbm.at[1, pl.ds(0, GROUP)],
                              dst_st.at[b], sems.at[2 * b]).wait()
        pltpu.make_async_copy(w_hbm.at[pl.ds(0, GROUP)],
                              w_st.at[b], sems.at[2 * b]).wait()

    @pl.when(wid == NC * NS - 1)
    def _():
        rows = pl.ds(TAILB, EROWS - TAILB)
        pltpu.sync_copy(ei_hbm.at[0, rows], src_st.at[0, pl.ds(0, 2)])
        pltpu.sync_copy(ei_hbm.at[1, rows], dst_st.at[0, pl.ds(0, 2)])
        pltpu.sync_copy(w_hbm.at[rows], w_st.at[0, pl.ds(0, 2)])
        for j in range(EROWS - TAILB):
            for k in range(ROW_W // LANES):
                sl = pl.ds(k * LANES, LANES)
                idx = src_st[0, j, sl]
                xg = plsc.load_gather(xv, [idx])
                msg_st[0, j, sl] = xg * w_st[0, j, sl]
            pltpu.sync_copy(msg_st.at[0, j], acc.at[dst_st.at[0, j]], add=True)

    plsc.subcore_barrier()
    osl = pl.ds(s * SLICE, SLICE)
    pltpu.sync_copy(acc.at[osl], vbuf)

    @pl.when(c == 0)
    def _():
        pltpu.sync_copy(vbuf, outa_hbm.at[osl])

    @pl.when(c == 1)
    def _():
        pltpu.sync_copy(vbuf, outb_hbm.at[osl])


def _edge_pass2(s1a_hbm, s1b_hbm, ei_hbm, w_hbm,
                sua_hbm, sub_hbm, sva_hbm, svb_hbm,
                t0, t1, src_st, dst_st, w_st, mu_st, mv_st, vbuf,
                accu, accv, sems):
    c = lax.axis_index("c")
    s = lax.axis_index("s")
    wid = c * NS + s
    pltpu.sync_copy(s1a_hbm, t0)
    pltpu.sync_copy(s1b_hbm, t1)
    _zero_fill(vbuf)
    pltpu.sync_copy(vbuf, accu.at[pl.ds(s * SLICE, SLICE)])
    pltpu.sync_copy(vbuf, accv.at[pl.ds(s * SLICE, SLICE)])
    plsc.subcore_barrier()

    def stage(r, b):
        base, vmask = _group_base(wid, r)
        rows = pl.ds(base, GROUP)
        cps = (pltpu.async_copy(ei_hbm.at[0, rows], src_st.at[b], sems.at[2 * b]),
               pltpu.async_copy(ei_hbm.at[1, rows], dst_st.at[b], sems.at[2 * b]),
               pltpu.async_copy(w_hbm.at[rows], w_st.at[b], sems.at[2 * b]))
        return cps, vmask

    def fire(b, vmask):
        cps = []
        for j in range(GROUP):
            for k in range(ROW_W // LANES):
                sl = pl.ds(k * LANES, LANES)
                idx = src_st[b, j, sl]
                sval = plsc.load_gather(t0, [idx]) + plsc.load_gather(t1, [idx])
                wv = w_st[b, j, sl] * vmask
                mu_st[b, j, sl] = jnp.maximum(sval, 0.0) * wv
                mv_st[b, j, sl] = jnp.maximum(-sval, 0.0) * wv
            cps.append(pltpu.async_copy(
                mu_st.at[b, j], accu.at[dst_st.at[b, j]],
                sems.at[2 * b + 1], add=True))
            cps.append(pltpu.async_copy(
                mv_st.at[b, j], accv.at[dst_st.at[b, j]],
                sems.at[2 * b + 1], add=True))
        return cps

    st0, m0 = stage(0, 0)
    st1, m1 = stage(1, 1)

    def body(i, carry):
        m0, m1 = carry
        r = 2 * i
        for cp in st0:
            cp.wait()
        sc0 = fire(0, m0)
        for cp in st1:
            cp.wait()
        sc1 = fire(1, m1)
        for cp in sc0:
            cp.wait()
        _, nm0 = stage(r + 2, 0)
        for cp in sc1:
            cp.wait()
        _, nm1 = stage(r + 3, 1)
        return (nm0, nm1)

    lax.fori_loop(0, ROUNDS, body, (m0, m1))
    for b in range(2):
        pltpu.make_async_copy(ei_hbm.at[0, pl.ds(0, GROUP)],
                              src_st.at[b], sems.at[2 * b]).wait()
        pltpu.make_async_copy(ei_hbm.at[1, pl.ds(0, GROUP)],
                              dst_st.at[b], sems.at[2 * b]).wait()
        pltpu.make_async_copy(w_hbm.at[pl.ds(0, GROUP)],
                              w_st.at[b], sems.at[2 * b]).wait()

    @pl.when(wid == NC * NS - 1)
    def _():
        rows = pl.ds(TAILB, EROWS - TAILB)
        pltpu.sync_copy(ei_hbm.at[0, rows], src_st.at[0, pl.ds(0, 2)])
        pltpu.sync_copy(ei_hbm.at[1, rows], dst_st.at[0, pl.ds(0, 2)])
        pltpu.sync_copy(w_hbm.at[rows], w_st.at[0, pl.ds(0, 2)])
        for j in range(EROWS - TAILB):
            for k in range(ROW_W // LANES):
                sl = pl.ds(k * LANES, LANES)
                idx = src_st[0, j, sl]
                sval = plsc.load_gather(t0, [idx]) + plsc.load_gather(t1, [idx])
                wv = w_st[0, j, sl]
                mu_st[0, j, sl] = jnp.maximum(sval, 0.0) * wv
                mv_st[0, j, sl] = jnp.maximum(-sval, 0.0) * wv
            pltpu.sync_copy(mu_st.at[0, j], accu.at[dst_st.at[0, j]], add=True)
            pltpu.sync_copy(mv_st.at[0, j], accv.at[dst_st.at[0, j]], add=True)

    plsc.subcore_barrier()
    osl = pl.ds(s * SLICE, SLICE)
    pltpu.sync_copy(accu.at[osl], vbuf)

    @pl.when(c == 0)
    def _():
        pltpu.sync_copy(vbuf, sua_hbm.at[osl])

    @pl.when(c == 1)
    def _():
        pltpu.sync_copy(vbuf, sub_hbm.at[osl])

    pltpu.sync_copy(accv.at[osl], vbuf)

    @pl.when(c == 0)
    def _():
        pltpu.sync_copy(vbuf, sva_hbm.at[osl])

    @pl.when(c == 1)
    def _():
        pltpu.sync_copy(vbuf, svb_hbm.at[osl])


_pass1 = pl.kernel(
    _edge_pass1,
    out_type=(jax.ShapeDtypeStruct((NPAD,), jnp.float32),
              jax.ShapeDtypeStruct((NPAD,), jnp.float32)),
    mesh=_mesh,
    compiler_params=_sc_params,
    scratch_types=[
        pltpu.VMEM((N,), jnp.float32),
        pltpu.VMEM((2, GROUP, ROW_W), jnp.int32),
        pltpu.VMEM((2, GROUP, ROW_W), jnp.int32),
        pltpu.VMEM((2, GROUP, ROW_W), jnp.float32),
        pltpu.VMEM((2, GROUP, ROW_W), jnp.float32),
        pltpu.VMEM((SLICE,), jnp.float32),
        pltpu.VMEM_SHARED((NPAD,), jnp.float32),
        pltpu.SemaphoreType.DMA((4,)),
    ],
)

_pass2 = pl.kernel(
    _edge_pass2,
    out_type=(jax.ShapeDtypeStruct((NPAD,), jnp.float32),
              jax.ShapeDtypeStruct((NPAD,), jnp.float32),
              jax.ShapeDtypeStruct((NPAD,), jnp.float32),
              jax.ShapeDtypeStruct((NPAD,), jnp.float32)),
    mesh=_mesh,
    compiler_params=_sc_params,
    scratch_types=[
        pltpu.VMEM((NPAD,), jnp.float32),
        pltpu.VMEM((NPAD,), jnp.float32),
        pltpu.VMEM((2, GROUP, ROW_W), jnp.int32),
        pltpu.VMEM((2, GROUP, ROW_W), jnp.int32),
        pltpu.VMEM((2, GROUP, ROW_W), jnp.float32),
        pltpu.VMEM((2, GROUP, ROW_W), jnp.float32),
        pltpu.VMEM((2, GROUP, ROW_W), jnp.float32),
        pltpu.VMEM((SLICE,), jnp.float32),
        pltpu.VMEM_SHARED((NPAD,), jnp.float32),
        pltpu.VMEM_SHARED((NPAD,), jnp.float32),
        pltpu.SemaphoreType.DMA((4,)),
    ],
)


def _epilogue(sua_ref, sub_ref, sva_ref, svb_ref,
              w1_ref, w2_ref, b2_ref, wl_ref, bl_ref, out_ref):
    su = sua_ref[...] + sub_ref[...]
    sv = sva_ref[...] + svb_ref[...]
    w1 = w1_ref[...]
    w2 = w2_ref[...]
    alpha = jnp.dot(jnp.maximum(w1, 0.0), w2,
                    preferred_element_type=jnp.float32)
    beta = jnp.dot(jnp.maximum(-w1, 0.0), w2,
                   preferred_element_type=jnp.float32)
    rows = NPAD // 128
    i = (lax.broadcasted_iota(jnp.int32, (rows, 128), 0) * 128
         + lax.broadcasted_iota(jnp.int32, (rows, 128), 1))
    seg = (i * NSEG) // N
    gain = jnp.zeros((rows, 128), jnp.float32)
    for sgi in range(NSEG):
        cnt = -((-(sgi + 1) * N) // NSEG) - -((-sgi * N) // NSEG)
        gain = jnp.where(seg == sgi, wl_ref[sgi, 0] * (1.0 / cnt), gain)
    gain = jnp.where(i < N, gain, 0.0)
    lane = lax.broadcasted_iota(jnp.int32, (1, 128), 1)
    vec = jnp.zeros((1, 128), jnp.float32)
    for ci in range(32):
        h2c = jnp.maximum(su * alpha[0, ci] + sv * beta[0, ci] + b2_ref[0, ci],
                          0.0)
        vec = jnp.where(lane == ci, jnp.sum(h2c * gain), vec)
    out_ref[...] = 1.0 / (1.0 + jnp.exp(-(vec + bl_ref[0, 0])))


_epilogue_call = pl.pallas_call(
    _epilogue,
    out_shape=jax.ShapeDtypeStruct((1, 128), jnp.float32),
)


@jax.jit
def kernel(x, edge_index, edge_weight, W1, b1, W2, b2, Wl, bl):
    ei3 = edge_index.reshape(2, EROWS, ROW_W)
    w2e = edge_weight.reshape(EROWS, ROW_W)

    s1a, s1b = _pass1(x.reshape(N), ei3, w2e)
    sua, sub, sva, svb = _pass2(s1a, s1b, ei3, w2e)

    r = NPAD // 128
    vec = _epilogue_call(sua.reshape(r, 128), sub.reshape(r, 128),
                         sva.reshape(r, 128), svb.reshape(r, 128),
                         W1, W2, b2.reshape(1, 32), Wl, bl.reshape(1, 1))
    return vec[0, :32].reshape(32, 1)


# speedup vs baseline: 1.1016x; 1.0347x over previous
import functools
import math

import jax
import jax.numpy as jnp
from jax import lax
from jax.experimental import pallas as pl
from jax.experimental.pallas import tpu as pltpu
from jax.experimental.pallas import tpu_sc as plsc

N = 50000
E = 800000
NSEG = 14
NC = 2
NS = 16
LANES = 16

NPAD = 50176
ROW_W = 128
EROWS = E // ROW_W
GROUP = 8
FULLG = (EROWS // GROUP)
TAILB = FULLG * GROUP
ROUNDS = 13
SLICE = NPAD // NS

_mesh = plsc.VectorSubcoreMesh(core_axis_name="c", subcore_axis_name="s")
_sc_params = pltpu.CompilerParams(needs_layout_passes=False)


def _zero_fill(vbuf):
    zv = jnp.zeros((LANES,), jnp.float32)

    def zbody(i, carry):
        vbuf[pl.ds(i * LANES, LANES)] = zv
        return carry

    lax.fori_loop(0, SLICE // LANES, zbody, 0)


def _group_base(wid, r):
    gid = wid + (NC * NS) * r
    valid = gid * GROUP < TAILB
    base = pl.multiple_of(jnp.where(valid, gid, 0) * GROUP, GROUP)
    vmask = valid.astype(jnp.float32)
    return base, vmask


def _edge_pass1(x_hbm, ei_hbm, w_hbm, outa_hbm, outb_hbm,
                xv, src_st, dst_st, w_st, msg_st, vbuf, acc, sems):
    c = lax.axis_index("c")
    s = lax.axis_index("s")
    wid = c * NS + s
    xcp = pltpu.async_copy(x_hbm, xv, sems.at[4])
    _zero_fill(vbuf)
    pltpu.sync_copy(vbuf, acc.at[pl.ds(s * SLICE, SLICE)])
    plsc.subcore_barrier()

    def stage(r, b):
        base, vmask = _group_base(wid, r)
        rows = pl.ds(base, GROUP)
        cps = (pltpu.async_copy(ei_hbm.at[0, rows], src_st.at[b], sems.at[2 * b]),
               pltpu.async_copy(ei_hbm.at[1, rows], dst_st.at[b], sems.at[2 * b]),
               pltpu.async_copy(w_hbm.at[rows], w_st.at[b], sems.at[2 * b]))
        return cps, vmask

    def fire(b, vmask):
        cps = []
        for j in range(GROUP):
            for k in range(ROW_W // LANES):
                sl = pl.ds(k * LANES, LANES)
                idx = src_st[b, j, sl]
                xg = plsc.load_gather(xv, [idx])
                msg_st[b, j, sl] = xg * w_st[b, j, sl] * vmask
            cps.append(pltpu.async_copy(
                msg_st.at[b, j], acc.at[dst_st.at[b, j]],
                sems.at[2 * b + 1], add=True))
        return cps

    st0, m0 = stage(0, 0)
    st1, m1 = stage(1, 1)
    xcp.wait()

    def body(i, carry):
        m0, m1 = carry
        r = 2 * i
        for cp in st0:
            cp.wait()
        sc0 = fire(0, m0)
        for cp in st1:
            cp.wait()
        sc1 = fire(1, m1)
        for cp in sc0:
            cp.wait()
        _, nm0 = stage(r + 2, 0)
        for cp in sc1:
            cp.wait()
        _, nm1 = stage(r + 3, 1)
        return (nm0, nm1)

    lax.fori_loop(0, ROUNDS, body, (m0, m1))
    for b in range(2):
        pltpu.make_async_copy(ei_hbm.at[0, pl.ds(0, GROUP)],
                              src_st.at[b], sems.at[2 * b]).wait()
        pltpu.make_async_copy(ei_hbm.at[1, pl.ds(0, GROUP)],
                              dst_st.at[b], sems.at[2 * b]).wait()
        pltpu.make_async_copy(w_hbm.at[pl.ds(0, GROUP)],
                              w_st.at[b], sems.at[2 * b]).wait()

    @pl.when(wid == NC * NS - 1)
    def _():
        rows = pl.ds(TAILB, EROWS - TAILB)
        pltpu.sync_copy(ei_hbm.at[0, rows], src_st.at[0, pl.ds(0, 2)])
        pltpu.sync_copy(ei_hbm.at[1, rows], dst_st.at[0, pl.ds(0, 2)])
        pltpu.sync_copy(w_hbm.at[rows], w_st.at[0, pl.ds(0, 2)])
        for j in range(EROWS - TAILB):
            for k in range(ROW_W // LANES):
                sl = pl.ds(k * LANES, LANES)
                idx = src_st[0, j, sl]
                xg = plsc.load_gather(xv, [idx])
                msg_st[0, j, sl] = xg * w_st[0, j, sl]
            pltpu.sync_copy(msg_st.at[0, j], acc.at[dst_st.at[0, j]], add=True)

    plsc.subcore_barrier()
    osl = pl.ds(s * SLICE, SLICE)
    pltpu.sync_copy(acc.at[osl], vbuf)

    @pl.when(c == 0)
    def _():
        pltpu.sync_copy(vbuf, outa_hbm.at[osl])

    @pl.when(c == 1)
    def _():
        pltpu.sync_copy(vbuf, outb_hbm.at[osl])


def _edge_pass2(s1a_hbm, s1b_hbm, ei_hbm, w_hbm,
                sua_hbm, sub_hbm, sva_hbm, svb_hbm,
                t0, t1, src_st, dst_st, w_st, mu_st, mv_st, vbuf,
                accu, accv, sems):
    c = lax.axis_index("c")
    s = lax.axis_index("s")
    wid = c * NS + s
    tcp0 = pltpu.async_copy(s1a_hbm, t0, sems.at[4])
    tcp1 = pltpu.async_copy(s1b_hbm, t1, sems.at[4])
    _zero_fill(vbuf)
    pltpu.sync_copy(vbuf, accu.at[pl.ds(s * SLICE, SLICE)])
    pltpu.sync_copy(vbuf, accv.at[pl.ds(s * SLICE, SLICE)])
    plsc.subcore_barrier()

    def stage(r, b):
        base, vmask = _group_base(wid, r)
        rows = pl.ds(base, GROUP)
        cps = (pltpu.async_copy(ei_hbm.at[0, rows], src_st.at[b], sems.at[2 * b]),
               pltpu.async_copy(ei_hbm.at[1, rows], dst_st.at[b], sems.at[2 * b]),
               pltpu.async_copy(w_hbm.at[rows], w_st.at[b], sems.at[2 * b]))
        return cps, vmask

    def fire(b, vmask):
        cps = []
        for j in range(GROUP):
            for k in range(ROW_W // LANES):
                sl = pl.ds(k * LANES, LANES)
                idx = src_st[b, j, sl]
                sval = plsc.load_gather(t0, [idx]) + plsc.load_gather(t1, [idx])
                wv = w_st[b, j, sl] * vmask
                mu_st[b, j, sl] = jnp.maximum(sval, 0.0) * wv
                mv_st[b, j, sl] = jnp.maximum(-sval, 0.0) * wv
            cps.append(pltpu.async_copy(
                mu_st.at[b, j], accu.at[dst_st.at[b, j]],
                sems.at[2 * b + 1], add=True))
            cps.append(pltpu.async_copy(
                mv_st.at[b, j], accv.at[dst_st.at[b, j]],
                sems.at[2 * b + 1], add=True))
        return cps

    st0, m0 = stage(0, 0)
    st1, m1 = stage(1, 1)
    tcp0.wait()
    tcp1.wait()

    def body(i, carry):
        m0, m1 = carry
        r = 2 * i
        for cp in st0:
            cp.wait()
        sc0 = fire(0, m0)
        for cp in st1:
            cp.wait()
        sc1 = fire(1, m1)
        for cp in sc0:
            cp.wait()
        _, nm0 = stage(r + 2, 0)
        for cp in sc1:
            cp.wait()
        _, nm1 = stage(r + 3, 1)
        return (nm0, nm1)

    lax.fori_loop(0, ROUNDS, body, (m0, m1))
    for b in range(2):
        pltpu.make_async_copy(ei_hbm.at[0, pl.ds(0, GROUP)],
                              src_st.at[b], sems.at[2 * b]).wait()
        pltpu.make_async_copy(ei_hbm.at[1, pl.ds(0, GROUP)],
                              dst_st.at[b], sems.at[2 * b]).wait()
        pltpu.make_async_copy(w_hbm.at[pl.ds(0, GROUP)],
                              w_st.at[b], sems.at[2 * b]).wait()

    @pl.when(wid == NC * NS - 1)
    def _():
        rows = pl.ds(TAILB, EROWS - TAILB)
        pltpu.sync_copy(ei_hbm.at[0, rows], src_st.at[0, pl.ds(0, 2)])
        pltpu.sync_copy(ei_hbm.at[1, rows], dst_st.at[0, pl.ds(0, 2)])
        pltpu.sync_copy(w_hbm.at[rows], w_st.at[0, pl.ds(0, 2)])
        for j in range(EROWS - TAILB):
            for k in range(ROW_W // LANES):
                sl = pl.ds(k * LANES, LANES)
                idx = src_st[0, j, sl]
                sval = plsc.load_gather(t0, [idx]) + plsc.load_gather(t1, [idx])
                wv = w_st[0, j, sl]
                mu_st[0, j, sl] = jnp.maximum(sval, 0.0) * wv
                mv_st[0, j, sl] = jnp.maximum(-sval, 0.0) * wv
            pltpu.sync_copy(mu_st.at[0, j], accu.at[dst_st.at[0, j]], add=True)
            pltpu.sync_copy(mv_st.at[0, j], accv.at[dst_st.at[0, j]], add=True)

    plsc.subcore_barrier()
    osl = pl.ds(s * SLICE, SLICE)
    pltpu.sync_copy(accu.at[osl], vbuf)

    @pl.when(c == 0)
    def _():
        pltpu.sync_copy(vbuf, sua_hbm.at[osl])

    @pl.when(c == 1)
    def _():
        pltpu.sync_copy(vbuf, sub_hbm.at[osl])

    pltpu.sync_copy(accv.at[osl], vbuf)

    @pl.when(c == 0)
    def _():
        pltpu.sync_copy(vbuf, sva_hbm.at[osl])

    @pl.when(c == 1)
    def _():
        pltpu.sync_copy(vbuf, svb_hbm.at[osl])


_pass1 = pl.kernel(
    _edge_pass1,
    out_type=(jax.ShapeDtypeStruct((NPAD,), jnp.float32),
              jax.ShapeDtypeStruct((NPAD,), jnp.float32)),
    mesh=_mesh,
    compiler_params=_sc_params,
    scratch_types=[
        pltpu.VMEM((N,), jnp.float32),
        pltpu.VMEM((2, GROUP, ROW_W), jnp.int32),
        pltpu.VMEM((2, GROUP, ROW_W), jnp.int32),
        pltpu.VMEM((2, GROUP, ROW_W), jnp.float32),
        pltpu.VMEM((2, GROUP, ROW_W), jnp.float32),
        pltpu.VMEM((SLICE,), jnp.float32),
        pltpu.VMEM_SHARED((NPAD,), jnp.float32),
        pltpu.SemaphoreType.DMA((5,)),
    ],
)

_pass2 = pl.kernel(
    _edge_pass2,
    out_type=(jax.ShapeDtypeStruct((NPAD,), jnp.float32),
              jax.ShapeDtypeStruct((NPAD,), jnp.float32),
              jax.ShapeDtypeStruct((NPAD,), jnp.float32),
              jax.ShapeDtypeStruct((NPAD,), jnp.float32)),
    mesh=_mesh,
    compiler_params=_sc_params,
    scratch_types=[
        pltpu.VMEM((NPAD,), jnp.float32),
        pltpu.VMEM((NPAD,), jnp.float32),
        pltpu.VMEM((2, GROUP, ROW_W), jnp.int32),
        pltpu.VMEM((2, GROUP, ROW_W), jnp.int32),
        pltpu.VMEM((2, GROUP, ROW_W), jnp.float32),
        pltpu.VMEM((2, GROUP, ROW_W), jnp.float32),
        pltpu.VMEM((2, GROUP, ROW_W), jnp.float32),
        pltpu.VMEM((SLICE,), jnp.float32),
        pltpu.VMEM_SHARED((NPAD,), jnp.float32),
        pltpu.VMEM_SHARED((NPAD,), jnp.float32),
        pltpu.SemaphoreType.DMA((5,)),
    ],
)


def _epilogue(sua_ref, sub_ref, sva_ref, svb_ref,
              w1_ref, w2_ref, b2_ref, wl_ref, bl_ref, out_ref):
    su = sua_ref[...] + sub_ref[...]
    sv = sva_ref[...] + svb_ref[...]
    w1 = w1_ref[...]
    w2 = w2_ref[...]
    alpha = jnp.dot(jnp.maximum(w1, 0.0), w2,
                    preferred_element_type=jnp.float32)
    beta = jnp.dot(jnp.maximum(-w1, 0.0), w2,
                   preferred_element_type=jnp.float32)
    rows = NPAD // 128
    i = (lax.broadcasted_iota(jnp.int32, (rows, 128), 0) * 128
         + lax.broadcasted_iota(jnp.int32, (rows, 128), 1))
    seg = (i * NSEG) // N
    gain = jnp.zeros((rows, 128), jnp.float32)
    for sgi in range(NSEG):
        cnt = -((-(sgi + 1) * N) // NSEG) - -((-sgi * N) // NSEG)
        gain = jnp.where(seg == sgi, wl_ref[sgi, 0] * (1.0 / cnt), gain)
    gain = jnp.where(i < N, gain, 0.0)
    lane = lax.broadcasted_iota(jnp.int32, (1, 128), 1)
    vec = jnp.zeros((1, 128), jnp.float32)
    for ci in range(32):
        h2c = jnp.maximum(su * alpha[0, ci] + sv * beta[0, ci] + b2_ref[0, ci],
                          0.0)
        vec = jnp.where(lane == ci, jnp.sum(h2c * gain), vec)
    out_ref[...] = 1.0 / (1.0 + jnp.exp(-(vec + bl_ref[0, 0])))


_epilogue_call = pl.pallas_call(
    _epilogue,
    out_shape=jax.ShapeDtypeStruct((1, 128), jnp.float32),
)


@jax.jit
def kernel(x, edge_index, edge_weight, W1, b1, W2, b2, Wl, bl):
    ei3 = edge_index.reshape(2, EROWS, ROW_W)
    w2e = edge_weight.reshape(EROWS, ROW_W)

    s1a, s1b = _pass1(x.reshape(N), ei3, w2e)
    sua, sub, sva, svb = _pass2(s1a, s1b, ei3, w2e)

    r = NPAD // 128
    vec = _epilogue_call(sua.reshape(r, 128), sub.reshape(r, 128),
                         sva.reshape(r, 128), svb.reshape(r, 128),
                         W1, W2, b2.reshape(1, 32), Wl, bl.reshape(1, 1))
    return vec[0, :32].reshape(32, 1)
